# Initial kernel scaffold; baseline (speedup 1.0000x reference)
import jax
import jax.numpy as jnp
from jax.experimental import pallas as pl


def kernel(fmap, weight, bias):
    raise NotImplementedError("write your pallas kernel here")


# speedup vs baseline: 5.7050x; 5.7050x over previous
import functools

import jax
import jax.numpy as jnp
import numpy as np
from jax import lax
from jax.experimental import pallas as pl
from jax.experimental.pallas import tpu as pltpu
from jax.experimental.pallas import tpu_sc as plsc

IN_C = 96
OUT_C = 192
H = 224
W = 224
L = H * W
PH = 226
PW = 228
FLAT = PH * PW
MBLK = 512
NBLK = (FLAT + MBLK - 1) // MBLK
FLATP = NBLK * MBLK
XOFF = 232
XE = 52224
BUCKET_BITS = 16
BOFF = 1 << (BUCKET_BITS - 1)
NBUCK = 1 << BUCKET_BITS
NT = 16
TLOC = NBUCK // NT
CH = L // NT
CPT = OUT_C // NT
OCH = L // 4

_SCALE = np.float32(10000.0) / np.float32(864.0)


def _summary_body(x_ref, b_ref, s_acc):
    k = pl.program_id(1)
    P = [jnp.zeros((H, W), jnp.float32) for _ in range(8)]
    for ci in range(32):
        for kk in range(9):
            ki, kj = kk // 3, kk % 3
            r = (ci + kk) % 8
            P[r] = P[r] + x_ref[0, ci, ki:ki + H, kj:kj + W]
    R = ((P[0] + P[4]) + (P[2] + P[6])) + ((P[1] + P[5]) + (P[3] + P[7]))

    @pl.when(k == 0)
    def _():
        s_acc[...] = R

    @pl.when(k > 0)
    def _():
        s_acc[...] = s_acc[...] + R

    @pl.when(k == 2)
    def _():
        s = (s_acc[...] * _SCALE).astype(jnp.int32)
        b_ref[0] = jnp.clip(s + BOFF, 0, NBUCK - 1)


def _conv_body(xe_ref, w_ref, bias_ref, o_ref):
    j = pl.program_id(1)
    xsup = xe_ref[0, :, pl.ds(j * MBLK, 2 * MBLK)]
    acc = jnp.zeros((OUT_C, MBLK), jnp.float32)
    for kk in range(9):
        ki, kj = kk // 3, kk % 3
        off = XOFF + (ki - 1) * PW + (kj - 1)
        xblk = lax.slice(xsup, (0, off), (IN_C, off + MBLK))
        acc = acc + lax.dot_general(
            w_ref[kk], xblk, (((1,), (0,)), ((), ())),
            preferred_element_type=jnp.float32)
    o_ref[0] = acc + bias_ref[...]


def _winner_body(b_hbm, qwin_hbm, tbl_hbm, bloc, tloc, tfull, rbuf):
    n = lax.axis_index("c")
    t = lax.axis_index("s")
    pltpu.sync_copy(b_hbm.at[pl.ds(n * L, L)], bloc)

    lanes = lax.iota(jnp.int32, 16)

    def sbody(v, carry):
        row, colbase = carry
        bv = bloc[pl.ds(v * 16, 16)]
        own = (bv & (NT - 1)) == t
        loc = bv >> 4
        q = (row + 1) * PW + colbase + 1 + lanes
        plsc.store_scatter(tloc, [loc], q, mask=own)
        colbase = colbase + 16
        wrap = colbase >= W
        row = jnp.where(wrap, row + 1, row)
        colbase = jnp.where(wrap, 0, colbase)
        return row, colbase

    lax.fori_loop(0, L // 16, sbody,
                  (jnp.int32(0), jnp.int32(0)), unroll=4)

    pltpu.sync_copy(tloc, tbl_hbm.at[pl.ds(n * NBUCK + t * TLOC, TLOC)])
    plsc.subcore_barrier()
    pltpu.sync_copy(tbl_hbm.at[pl.ds(n * NBUCK, NBUCK)], tfull)

    base = t * CH

    def gbody(v, _):
        bv = bloc[pl.ds(base + v * 16, 16)]
        flat = (bv & (NT - 1)) * TLOC + (bv >> 4)
        rbuf[pl.ds(v * 16, 16)] = plsc.load_gather(tfull, [flat])
        return 0

    lax.fori_loop(0, CH // 16, gbody, 0, unroll=8)
    pltpu.sync_copy(rbuf, qwin_hbm.at[pl.ds(n * L + base, CH)])


def _route_body(conv_hbm, qwin_hbm, out_hbm, qloc, row, obuf):
    n = lax.axis_index("c")
    t = lax.axis_index("s")
    pltpu.sync_copy(qwin_hbm.at[pl.ds(n * L, L)], qloc)
    for ci in range(CPT):
        c = t * CPT + ci
        pltpu.sync_copy(conv_hbm.at[pl.ds((n * OUT_C + c) * FLATP, FLATP)], row)
        for ch in range(4):
            cbase = ch * OCH

            def gb(v, _):
                q = qloc[pl.ds(cbase + v * 16, 16)]
                obuf[pl.ds(v * 16, 16)] = plsc.load_gather(row, [q])
                return 0

            lax.fori_loop(0, OCH // 16, gb, 0, unroll=8)
            pltpu.sync_copy(
                obuf, out_hbm.at[pl.ds((n * OUT_C + c) * L + cbase, OCH)])


@functools.cache
def _sc_kernels():
    mesh = plsc.VectorSubcoreMesh(core_axis_name="c", subcore_axis_name="s")
    winner = functools.partial(
        pl.kernel, mesh=mesh,
        out_type=(
            jax.ShapeDtypeStruct((2 * L,), jnp.int32),
            jax.ShapeDtypeStruct((2 * NBUCK,), jnp.int32),
        ),
        scratch_types=[
            pltpu.VMEM((L,), jnp.int32),
            pltpu.VMEM((TLOC,), jnp.int32),
            pltpu.VMEM((NBUCK,), jnp.int32),
            pltpu.VMEM((CH,), jnp.int32),
        ],
        compiler_params=pltpu.CompilerParams(needs_layout_passes=False),
    )(_winner_body)
    route = functools.partial(
        pl.kernel, mesh=mesh,
        out_type=jax.ShapeDtypeStruct((2 * OUT_C * L,), jnp.float32),
        scratch_types=[
            pltpu.VMEM((L,), jnp.int32),
            pltpu.VMEM((FLATP,), jnp.float32),
            pltpu.VMEM((OCH,), jnp.float32),
        ],
        compiler_params=pltpu.CompilerParams(needs_layout_passes=False),
    )(_route_body)
    return winner, route


def kernel(fmap, weight, bias):
    N = fmap.shape[0]
    _winner_kernel, _route_kernel = _sc_kernels()
    padded = jnp.pad(fmap, ((0, 0), (0, 0), (1, 1), (1, 3)))

    b = pl.pallas_call(
        _summary_body,
        grid=(N, 3),
        in_specs=[pl.BlockSpec((1, 32, PH, PW), lambda n, k: (n, k, 0, 0))],
        out_specs=pl.BlockSpec((1, H, W), lambda n, k: (n, 0, 0)),
        out_shape=jax.ShapeDtypeStruct((N, H, W), jnp.int32),
        scratch_shapes=[pltpu.VMEM((H, W), jnp.float32)],
    )(padded)
    b = b.reshape(N * L)

    qwin, _ = _winner_kernel(b)

    xflat = padded.reshape(N, IN_C, FLAT)
    xe = jnp.pad(xflat, ((0, 0), (0, 0), (XOFF, XE - FLAT - XOFF)))
    wr = weight.reshape(OUT_C, IN_C, 9).transpose(2, 0, 1)

    convp = pl.pallas_call(
        _conv_body,
        grid=(N, NBLK),
        in_specs=[
            pl.BlockSpec((1, IN_C, XE), lambda n, j: (n, 0, 0)),
            pl.BlockSpec((9, OUT_C, IN_C), lambda n, j: (0, 0, 0)),
            pl.BlockSpec((OUT_C, 1), lambda n, j: (0, 0)),
        ],
        out_specs=pl.BlockSpec((1, OUT_C, MBLK), lambda n, j: (n, 0, j)),
        out_shape=jax.ShapeDtypeStruct((N, OUT_C, FLATP), jnp.float32),
    )(xe, wr, bias.reshape(OUT_C, 1))

    out = _route_kernel(convp.reshape(N * OUT_C * FLATP), qwin)
    return out.reshape(N, OUT_C, H, W)
